# Initial kernel scaffold; baseline (speedup 1.0000x reference)
import jax
import jax.numpy as jnp
from jax.experimental import pallas as pl


def kernel(role_labels, summar_role_embedding, token_embedding, entities_embedding, token_mask, entity_mask, pos2entity, char2token, entity2token, W_single, b_single, W_multi, b_multi, W_answer, b_answer):
    raise NotImplementedError("write your pallas kernel here")


# speedup vs baseline: 5.9242x; 5.9242x over previous
import functools

import jax
import jax.numpy as jnp
from jax import lax
from jax.experimental import pallas as pl
from jax.experimental.pallas import tpu as pltpu
from jax.experimental.pallas import tpu_sc as plsc

_R, _B, _S, _H, _NE = 8, 8, 2048, 256, 64
_L = 16
_NCH = _S // _L


def _k1_body(c_ref, w_ref, sr_ref, tok_ref, ent_ref, as_ref, am_ref, g_ref,
             tsm_ref):
    r = pl.program_id(1)
    dn = (((1,), (0,)), ((), ()))

    @pl.when(r == 0)
    def _():
        tok = tok_ref[0]
        ent = ent_ref[0]
        tk = lax.dot_general(tok, w_ref[:, 3:18], dn,
                             preferred_element_type=jnp.float32)
        em = lax.dot_general(ent, w_ref[:, 2:3], dn,
                             preferred_element_type=jnp.float32)
        g_ref[0, 0] = jnp.concatenate([tk, em], axis=1)
        tsm_ref[0, :] = tk[:, 0]
        tsm_ref[1, :] = em[:, 0]

    sr = sr_ref[0, 0]
    d = lax.dot_general(sr, w_ref[:, 0:2], dn,
                        preferred_element_type=jnp.float32)
    as_ref[0, 0, 0, :] = d[:, 0] + tsm_ref[0, :] + c_ref[r, 0]
    am_ref[0, 0, 0, :] = d[:, 1] + tsm_ref[1, :] + c_ref[r, 1]


def _run_k1(c, w, sr, tok, ent):
    f32 = jnp.float32
    return pl.pallas_call(
        _k1_body,
        grid=(_B, _R),
        in_specs=[
            pl.BlockSpec(memory_space=pltpu.SMEM),
            pl.BlockSpec((_H, 18), lambda b, r: (0, 0)),
            pl.BlockSpec((1, 1, _S, _H), lambda b, r: (r, b, 0, 0)),
            pl.BlockSpec((1, _S, _H), lambda b, r: (b, 0, 0)),
            pl.BlockSpec((1, _S, _H), lambda b, r: (b, 0, 0)),
        ],
        out_specs=[
            pl.BlockSpec((1, 1, 1, _S), lambda b, r: (b, r, 0, 0)),
            pl.BlockSpec((1, 1, 1, _S), lambda b, r: (b, r, 0, 0)),
            pl.BlockSpec((1, 1, _S, 16), lambda b, r: (b, 0, 0, 0)),
        ],
        out_shape=[
            jax.ShapeDtypeStruct((_B, _R, 1, _S), f32),
            jax.ShapeDtypeStruct((_B, _R, 1, _S), f32),
            jax.ShapeDtypeStruct((_B, 1, _S, 16), f32),
        ],
        scratch_shapes=[pltpu.VMEM((2, _S), f32)],
    )(c, w, sr, tok, ent)


def _sc_body(as_h, am_h, gs_h, gm_h, c2t_h, e2t_h, p2e_h, lab_h,
             merged_o, lab_o, max_o, se_o,
             as_v, am_v, gs_v, gm_v, emb_v, buf_s, buf_m, tokacc, entacc,
             merged_v, c2t_v, e2t_v, p2e_v, lab_v, acc3_v, red_v):
    cid = lax.axis_index("c")
    sid = lax.axis_index("s")

    iota = lax.iota(jnp.int32, _L)

    def allreduce(v, op):
        for k in (8, 4, 2, 1):
            red_v[...] = v
            v = op(v, plsc.load_gather(red_v, [iota ^ k]))
        return v

    @pl.when(jnp.logical_and(cid == 0, sid < _B))
    def _():
        b = sid
        pltpu.sync_copy(as_h.at[b], as_v)
        pltpu.sync_copy(am_h.at[b], am_v)
        pltpu.sync_copy(gs_h.at[b], gs_v)
        pltpu.sync_copy(gm_h.at[b], gm_v)
        pltpu.sync_copy(c2t_h.at[b], c2t_v)
        pltpu.sync_copy(e2t_h.at[b], e2t_v)
        pltpu.sync_copy(p2e_h.at[b], p2e_v)
        lab_v[...] = jnp.zeros((_L,), jnp.int32)
        pltpu.sync_copy(lab_h.at[pl.ds(pl.multiple_of(b * _R, 8), _R)],
                        lab_v.at[pl.ds(0, _R)])

        lab_idx = lab_v[...]
        z16 = jnp.zeros((_L,), jnp.float32)
        ninf = jnp.full((_L,), -jnp.inf, jnp.float32)
        acc_lab = z16
        acc_max = z16
        acc_se = z16

        for r in range(_R):
            def zero_tok(i, _):
                tokacc[pl.ds(pl.multiple_of(i * _L, _L), _L)] = z16
                return 0
            lax.fori_loop(0, _NCH, zero_tok, 0)
            for q in range(_NE // _L):
                entacc[pl.ds(q * _L, _L)] = z16

            def p1(i, carry):
                ms, mm = carry
                dsl = pl.ds(pl.multiple_of(i * _L, _L), _L)
                vs = as_v[r, dsl]
                vm = am_v[r, dsl]
                for j in range(r):
                    e = emb_v[j, dsl]
                    vs = vs + e * gs_v[r - 1 - j, dsl]
                    vm = vm + e * gm_v[r - 1 - j, dsl]
                buf_s[dsl] = vs
                buf_m[dsl] = vm
                return jnp.maximum(ms, vs), jnp.maximum(mm, vm)
            ms, mm = lax.fori_loop(0, _NCH, p1, (ninf, ninf))
            max_s = allreduce(ms, jnp.maximum)
            max_m = allreduce(mm, jnp.maximum)

            def p2(i, carry):
                ss, sm = carry
                dsl = pl.ds(pl.multiple_of(i * _L, _L), _L)
                es = jnp.exp(buf_s[dsl] - max_s)
                em = jnp.exp(buf_m[dsl] - max_m)
                buf_s[dsl] = es
                buf_m[dsl] = em
                return ss + es, sm + em
            ss, sm = lax.fori_loop(0, _NCH, p2, (z16, z16))
            inv_s = 1.0 / allreduce(ss, jnp.add)
            inv_m = 1.0 / allreduce(sm, jnp.add)

            def p3(i, _):
                dsl = pl.ds(pl.multiple_of(i * _L, _L), _L)
                plsc.addupdate_scatter(tokacc, [c2t_v[dsl]], buf_s[dsl] * inv_s)
                plsc.addupdate_scatter(entacc, [e2t_v[dsl]], buf_m[dsl] * inv_m)
                return 0
            lax.fori_loop(0, _NCH, p3, 0)

            def p4(i, m):
                dsl = pl.ds(pl.multiple_of(i * _L, _L), _L)
                t = tokacc[dsl]
                e = plsc.load_gather(entacc, [p2e_v[dsl]])
                mg = jnp.maximum(t, e)
                merged_v[dsl] = mg
                return jnp.maximum(m, mg)
            m2 = lax.fori_loop(0, _NCH, p4, ninf)
            max2 = allreduce(m2, jnp.maximum)

            def p5(i, sacc):
                dsl = pl.ds(pl.multiple_of(i * _L, _L), _L)
                return sacc + jnp.exp(merged_v[dsl] - max2)
            se2 = allreduce(lax.fori_loop(0, _NCH, p5, z16), jnp.add)

            if r < _R - 1:
                def p6(i, _):
                    dsl = pl.ds(pl.multiple_of(i * _L, _L), _L)
                    emb_v[r, dsl] = plsc.load_gather(merged_v, [c2t_v[dsl]])
                    return 0
                lax.fori_loop(0, _NCH, p6, 0)

            g = plsc.load_gather(merged_v, [lab_idx])
            sel = iota == r
            acc_lab = jnp.where(sel, g, acc_lab)
            acc_max = jnp.where(sel, max2, acc_max)
            acc_se = jnp.where(sel, se2, acc_se)

            pltpu.sync_copy(merged_v, merged_o.at[r, b])

        acc3_v[0] = acc_lab
        acc3_v[1] = acc_max
        acc3_v[2] = acc_se
        obase = pl.ds(pl.multiple_of(b * _L, 8), _L)
        pltpu.sync_copy(acc3_v.at[0], lab_o.at[obase])
        pltpu.sync_copy(acc3_v.at[1], max_o.at[obase])
        pltpu.sync_copy(acc3_v.at[2], se_o.at[obase])


def _run_sc(as_in, am_in, gs_in, gm_in, c2t, e2t, p2e, labs):
    f32 = jnp.float32
    i32 = jnp.int32
    mesh = plsc.VectorSubcoreMesh(core_axis_name="c", subcore_axis_name="s")
    fn = pl.kernel(
        _sc_body,
        mesh=mesh,
        compiler_params=pltpu.CompilerParams(needs_layout_passes=False),
        out_type=[
            jax.ShapeDtypeStruct((_R, _B, _S), f32),
            jax.ShapeDtypeStruct((_B * _L,), f32),
            jax.ShapeDtypeStruct((_B * _L,), f32),
            jax.ShapeDtypeStruct((_B * _L,), f32),
        ],
        scratch_types=[
            pltpu.VMEM((_R, _S), f32),
            pltpu.VMEM((_R, _S), f32),
            pltpu.VMEM((_R - 1, _S), f32),
            pltpu.VMEM((_R - 1, _S), f32),
            pltpu.VMEM((_R - 1, _S), f32),
            pltpu.VMEM((_S,), f32),
            pltpu.VMEM((_S,), f32),
            pltpu.VMEM((_S,), f32),
            pltpu.VMEM((_NE,), f32),
            pltpu.VMEM((_S,), f32),
            pltpu.VMEM((_S,), i32),
            pltpu.VMEM((_S,), i32),
            pltpu.VMEM((_S,), i32),
            pltpu.VMEM((_L,), i32),
            pltpu.VMEM((3, _L), f32),
            pltpu.VMEM((_L,), f32),
        ],
    )
    return fn(as_in, am_in, gs_in, gm_in, c2t, e2t, p2e, labs)


def _k3_body(lab_ref, max_ref, se_ref, mask_ref, out_ref):
    lp = lab_ref[...] - max_ref[...] - jnp.log(se_ref[...])
    cols = lax.broadcasted_iota(jnp.int32, (_B, _L), 1)
    lsum = jnp.sum(jnp.where(cols < _R, lp, 0.0))
    msum = jnp.sum(mask_ref[...])
    out_ref[...] = jnp.broadcast_to(-(lsum / _B) * msum, (1, 1))


def _run_k3(lab3, mx3, se3, mask):
    return pl.pallas_call(
        _k3_body,
        out_shape=jax.ShapeDtypeStruct((1, 1), jnp.float32),
    )(lab3, mx3, se3, mask)


def kernel(role_labels, summar_role_embedding, token_embedding,
           entities_embedding, token_mask, entity_mask, pos2entity,
           char2token, entity2token, W_single, b_single, W_multi, b_multi,
           W_answer, b_answer):
    f32 = jnp.float32
    i32 = jnp.int32

    ws1, ws2, ws3 = W_single[0:_H, 0], W_single[_H:2 * _H, 0], W_single[2 * _H:, 0]
    wm1, wm2, wm3 = W_multi[0:_H, 0], W_multi[_H:2 * _H, 0], W_multi[2 * _H:, 0]
    wa1, wa2 = W_answer[0:_H], W_answer[_H:2 * _H]

    vs, vm, cs, cm = [], [], [], []
    us, um = ws3, wm3
    acc_s = b_single[0].astype(f32)
    acc_m = b_multi[0].astype(f32)
    for _ in range(_R):
        cs.append(acc_s)
        cm.append(acc_m)
        vs.append(wa1 @ us)
        vm.append(wa1 @ um)
        acc_s = acc_s + b_answer @ us
        acc_m = acc_m + b_answer @ um
        us = wa2 @ us
        um = wa2 @ um

    w = jnp.stack([ws1, wm1, wm2, ws2] + vs[:_R - 1] + vm[:_R - 1], axis=1)
    c = jnp.stack([jnp.stack(cs), jnp.stack(cm)], axis=1).astype(f32)

    as4, am4, gall = _run_k1(c, w.astype(f32),
                             summar_role_embedding, token_embedding,
                             entities_embedding)
    as_in = as4.reshape(_B, _R, _S)
    am_in = am4.reshape(_B, _R, _S)
    gal = gall.reshape(_B, _S, 16).transpose(0, 2, 1)
    gs_in = gal[:, 1:_R]
    gm_in = gal[:, _R:2 * _R - 1]

    merged, lab3, mx3, se3 = _run_sc(
        as_in, am_in, gs_in, gm_in,
        char2token.astype(i32), entity2token.astype(i32),
        pos2entity.astype(i32), role_labels.T.astype(i32).reshape(-1))

    loss = _run_k3(lab3.reshape(_B, _L), mx3.reshape(_B, _L),
                   se3.reshape(_B, _L), token_mask.astype(f32))
    return loss[0, 0], merged


# speedup vs baseline: 8.1357x; 1.3733x over previous
import functools

import jax
import jax.numpy as jnp
from jax import lax
from jax.experimental import pallas as pl
from jax.experimental.pallas import tpu as pltpu
from jax.experimental.pallas import tpu_sc as plsc

_R, _B, _S, _H, _NE = 8, 8, 2048, 256, 64
_L = 16
_NCH = _S // _L


def _k1_body(c_ref, w_ref, sr_ref, tok_ref, ent_ref, as_ref, am_ref, g_ref,
             tsm_ref):
    r = pl.program_id(1)
    dn = (((0,), (1,)), ((), ()))

    @pl.when(r == 0)
    def _():
        tok = tok_ref[0]
        ent = ent_ref[0]
        tk = lax.dot_general(w_ref[:, 3:18], tok, dn,
                             preferred_element_type=jnp.float32)
        em = lax.dot_general(w_ref[:, 2:3], ent, dn,
                             preferred_element_type=jnp.float32)
        g_ref[0, 0:15, 0, :] = tk
        g_ref[0, 15:16, 0, :] = em
        tsm_ref[0:1, :] = tk[0:1, :]
        tsm_ref[1:2, :] = em

    sr = sr_ref[0, 0]
    d = lax.dot_general(w_ref[:, 0:2], sr, dn,
                        preferred_element_type=jnp.float32)
    as_ref[0, 0, 0, :] = d[0, :] + tsm_ref[0, :] + c_ref[r, 0]
    am_ref[0, 0, 0, :] = d[1, :] + tsm_ref[1, :] + c_ref[r, 1]


def _run_k1(c, w, sr, tok, ent):
    f32 = jnp.float32
    return pl.pallas_call(
        _k1_body,
        grid=(_B, _R),
        in_specs=[
            pl.BlockSpec(memory_space=pltpu.SMEM),
            pl.BlockSpec((_H, 18), lambda b, r: (0, 0)),
            pl.BlockSpec((1, 1, _S, _H), lambda b, r: (r, b, 0, 0)),
            pl.BlockSpec((1, _S, _H), lambda b, r: (b, 0, 0)),
            pl.BlockSpec((1, _S, _H), lambda b, r: (b, 0, 0)),
        ],
        out_specs=[
            pl.BlockSpec((1, 1, 1, _S), lambda b, r: (b, r, 0, 0)),
            pl.BlockSpec((1, 1, 1, _S), lambda b, r: (b, r, 0, 0)),
            pl.BlockSpec((1, 16, 1, _S), lambda b, r: (b, 0, 0, 0)),
        ],
        out_shape=[
            jax.ShapeDtypeStruct((_B, _R, 1, _S), f32),
            jax.ShapeDtypeStruct((_B, _R, 1, _S), f32),
            jax.ShapeDtypeStruct((_B, 16, 1, _S), f32),
        ],
        scratch_shapes=[pltpu.VMEM((2, _S), f32)],
    )(c, w, sr, tok, ent)


def _sc_body(as_h, am_h, gs_h, gm_h, c2t_h, e2t_h, p2e_h, lab_h,
             merged_o, lab_o, max_o, se_o,
             as_v, am_v, gs_v, gm_v, emb_v, buf_s, buf_m, tokacc, entacc,
             merged_v, c2t_v, e2t_v, p2e_v, lab_v, acc3_v, red_v):
    cid = lax.axis_index("c")
    sid = lax.axis_index("s")

    iota = lax.iota(jnp.int32, _L)

    def allreduce(v, op):
        for k in (8, 4, 2, 1):
            red_v[...] = v
            v = op(v, plsc.load_gather(red_v, [iota ^ k]))
        return v

    @pl.when(jnp.logical_and(cid == 0, sid < _B))
    def _():
        b = sid
        pltpu.sync_copy(as_h.at[b], as_v)
        pltpu.sync_copy(am_h.at[b], am_v)
        pltpu.sync_copy(gs_h.at[b], gs_v)
        pltpu.sync_copy(gm_h.at[b], gm_v)
        pltpu.sync_copy(c2t_h.at[b], c2t_v)
        pltpu.sync_copy(e2t_h.at[b], e2t_v)
        pltpu.sync_copy(p2e_h.at[b], p2e_v)
        lab_v[...] = jnp.zeros((_L,), jnp.int32)
        pltpu.sync_copy(lab_h.at[pl.ds(pl.multiple_of(b * _R, 8), _R)],
                        lab_v.at[pl.ds(0, _R)])

        lab_idx = lab_v[...]
        z16 = jnp.zeros((_L,), jnp.float32)
        ninf = jnp.full((_L,), -jnp.inf, jnp.float32)
        acc_lab = z16
        acc_max = z16
        acc_se = z16

        for r in range(_R):
            def zero_tok(i, _):
                tokacc[pl.ds(pl.multiple_of(i * _L, _L), _L)] = z16
                return 0
            lax.fori_loop(0, _NCH, zero_tok, 0)
            for q in range(_NE // _L):
                entacc[pl.ds(q * _L, _L)] = z16

            def p1(i, carry):
                ms, mm = carry
                dsl = pl.ds(pl.multiple_of(i * _L, _L), _L)
                vs = as_v[r, dsl]
                vm = am_v[r, dsl]
                for j in range(r):
                    e = emb_v[j, dsl]
                    vs = vs + e * gs_v[r - 1 - j, dsl]
                    vm = vm + e * gm_v[r - 1 - j, dsl]
                buf_s[dsl] = vs
                buf_m[dsl] = vm
                return jnp.maximum(ms, vs), jnp.maximum(mm, vm)
            ms, mm = lax.fori_loop(0, _NCH, p1, (ninf, ninf))
            max_s = allreduce(ms, jnp.maximum)
            max_m = allreduce(mm, jnp.maximum)

            def p2(i, carry):
                ss, sm = carry
                dsl = pl.ds(pl.multiple_of(i * _L, _L), _L)
                es = jnp.exp(buf_s[dsl] - max_s)
                em = jnp.exp(buf_m[dsl] - max_m)
                buf_s[dsl] = es
                buf_m[dsl] = em
                return ss + es, sm + em
            ss, sm = lax.fori_loop(0, _NCH, p2, (z16, z16))
            inv_s = 1.0 / allreduce(ss, jnp.add)
            inv_m = 1.0 / allreduce(sm, jnp.add)

            def p3(i, _):
                dsl = pl.ds(pl.multiple_of(i * _L, _L), _L)
                plsc.addupdate_scatter(tokacc, [c2t_v[dsl]], buf_s[dsl] * inv_s)
                plsc.addupdate_scatter(entacc, [e2t_v[dsl]], buf_m[dsl] * inv_m)
                return 0
            lax.fori_loop(0, _NCH, p3, 0)

            def p4(i, m):
                dsl = pl.ds(pl.multiple_of(i * _L, _L), _L)
                t = tokacc[dsl]
                e = plsc.load_gather(entacc, [p2e_v[dsl]])
                mg = jnp.maximum(t, e)
                merged_v[dsl] = mg
                return jnp.maximum(m, mg)
            m2 = lax.fori_loop(0, _NCH, p4, ninf)
            max2 = allreduce(m2, jnp.maximum)

            def p5(i, sacc):
                dsl = pl.ds(pl.multiple_of(i * _L, _L), _L)
                return sacc + jnp.exp(merged_v[dsl] - max2)
            se2 = allreduce(lax.fori_loop(0, _NCH, p5, z16), jnp.add)

            if r < _R - 1:
                def p6(i, _):
                    dsl = pl.ds(pl.multiple_of(i * _L, _L), _L)
                    emb_v[r, dsl] = plsc.load_gather(merged_v, [c2t_v[dsl]])
                    return 0
                lax.fori_loop(0, _NCH, p6, 0)

            g = plsc.load_gather(merged_v, [lab_idx])
            sel = iota == r
            acc_lab = jnp.where(sel, g, acc_lab)
            acc_max = jnp.where(sel, max2, acc_max)
            acc_se = jnp.where(sel, se2, acc_se)

            pltpu.sync_copy(merged_v, merged_o.at[r, b])

        acc3_v[0] = acc_lab
        acc3_v[1] = acc_max
        acc3_v[2] = acc_se
        obase = pl.ds(pl.multiple_of(b * _L, 8), _L)
        pltpu.sync_copy(acc3_v.at[0], lab_o.at[obase])
        pltpu.sync_copy(acc3_v.at[1], max_o.at[obase])
        pltpu.sync_copy(acc3_v.at[2], se_o.at[obase])


def _run_sc(as_in, am_in, gs_in, gm_in, c2t, e2t, p2e, labs):
    f32 = jnp.float32
    i32 = jnp.int32
    mesh = plsc.VectorSubcoreMesh(core_axis_name="c", subcore_axis_name="s")
    fn = pl.kernel(
        _sc_body,
        mesh=mesh,
        compiler_params=pltpu.CompilerParams(needs_layout_passes=False),
        out_type=[
            jax.ShapeDtypeStruct((_R, _B, _S), f32),
            jax.ShapeDtypeStruct((_B * _L,), f32),
            jax.ShapeDtypeStruct((_B * _L,), f32),
            jax.ShapeDtypeStruct((_B * _L,), f32),
        ],
        scratch_types=[
            pltpu.VMEM((_R, _S), f32),
            pltpu.VMEM((_R, _S), f32),
            pltpu.VMEM((_R - 1, _S), f32),
            pltpu.VMEM((_R - 1, _S), f32),
            pltpu.VMEM((_R - 1, _S), f32),
            pltpu.VMEM((_S,), f32),
            pltpu.VMEM((_S,), f32),
            pltpu.VMEM((_S,), f32),
            pltpu.VMEM((_NE,), f32),
            pltpu.VMEM((_S,), f32),
            pltpu.VMEM((_S,), i32),
            pltpu.VMEM((_S,), i32),
            pltpu.VMEM((_S,), i32),
            pltpu.VMEM((_L,), i32),
            pltpu.VMEM((3, _L), f32),
            pltpu.VMEM((_L,), f32),
        ],
    )
    return fn(as_in, am_in, gs_in, gm_in, c2t, e2t, p2e, labs)


def _k3_body(lab_ref, max_ref, se_ref, mask_ref, out_ref):
    lp = lab_ref[...] - max_ref[...] - jnp.log(se_ref[...])
    cols = lax.broadcasted_iota(jnp.int32, (_B, _L), 1)
    lsum = jnp.sum(jnp.where(cols < _R, lp, 0.0))
    msum = jnp.sum(mask_ref[...])
    out_ref[...] = jnp.broadcast_to(-(lsum / _B) * msum, (1, 1))


def _run_k3(lab3, mx3, se3, mask):
    return pl.pallas_call(
        _k3_body,
        out_shape=jax.ShapeDtypeStruct((1, 1), jnp.float32),
    )(lab3, mx3, se3, mask)


def kernel(role_labels, summar_role_embedding, token_embedding,
           entities_embedding, token_mask, entity_mask, pos2entity,
           char2token, entity2token, W_single, b_single, W_multi, b_multi,
           W_answer, b_answer):
    f32 = jnp.float32
    i32 = jnp.int32

    ws1, ws2, ws3 = W_single[0:_H, 0], W_single[_H:2 * _H, 0], W_single[2 * _H:, 0]
    wm1, wm2, wm3 = W_multi[0:_H, 0], W_multi[_H:2 * _H, 0], W_multi[2 * _H:, 0]
    wa1, wa2 = W_answer[0:_H], W_answer[_H:2 * _H]

    vs, vm, cs, cm = [], [], [], []
    us, um = ws3, wm3
    acc_s = b_single[0].astype(f32)
    acc_m = b_multi[0].astype(f32)
    for _ in range(_R):
        cs.append(acc_s)
        cm.append(acc_m)
        vs.append(wa1 @ us)
        vm.append(wa1 @ um)
        acc_s = acc_s + b_answer @ us
        acc_m = acc_m + b_answer @ um
        us = wa2 @ us
        um = wa2 @ um

    w = jnp.stack([ws1, wm1, wm2, ws2] + vs[:_R - 1] + vm[:_R - 1], axis=1)
    c = jnp.stack([jnp.stack(cs), jnp.stack(cm)], axis=1).astype(f32)

    as4, am4, gall = _run_k1(c, w.astype(f32),
                             summar_role_embedding, token_embedding,
                             entities_embedding)
    as_in = as4.reshape(_B, _R, _S)
    am_in = am4.reshape(_B, _R, _S)
    gal = gall.reshape(_B, 16, _S)
    gs_in = gal[:, 1:_R]
    gm_in = gal[:, _R:2 * _R - 1]

    merged, lab3, mx3, se3 = _run_sc(
        as_in, am_in, gs_in, gm_in,
        char2token.astype(i32), entity2token.astype(i32),
        pos2entity.astype(i32), role_labels.T.astype(i32).reshape(-1))

    loss = _run_k3(lab3.reshape(_B, _L), mx3.reshape(_B, _L),
                   se3.reshape(_B, _L), token_mask.astype(f32))
    return loss[0, 0], merged
